# baseline (speedup 1.0000x reference)
import functools

import jax
import jax.numpy as jnp
from jax.experimental import pallas as pl
from jax.experimental.pallas import tpu as pltpu

N = 10000
E = 160000
F = 128
H = 128
C = 16

_BLK = 1000


def _dense_body(x_ref, agg_ref, wrelT_ref, wrootT_ref, wmlpT_ref, brel_ref,
                bmlp_ref, s_ref, ssT_ref):
    i = pl.program_id(0)
    h = jnp.dot(agg_ref[...], wrelT_ref[...], preferred_element_type=jnp.float32)
    h = h + brel_ref[...]
    h = h + jnp.dot(x_ref[...], wrootT_ref[...], preferred_element_type=jnp.float32)
    h = jnp.maximum(h, 0.0)
    sl = jnp.dot(h, wmlpT_ref[...], preferred_element_type=jnp.float32) + bmlp_ref[...]
    m = jnp.max(sl, axis=-1, keepdims=True)
    e = jnp.exp(sl - m)
    p = e / jnp.sum(e, axis=-1, keepdims=True)
    s_ref[...] = p
    blk = jax.lax.dot_general(p, p, (((0,), (0,)), ((), ())),
                              preferred_element_type=jnp.float32)

    @pl.when(i == 0)
    def _():
        ssT_ref[...] = blk

    @pl.when(i != 0)
    def _():
        ssT_ref[...] += blk


def _dense_stage(x, agg, W1_rel, b1_rel, W1_root, W_mlp, b_mlp):
    grid = (N // _BLK,)
    full = lambda shape: pl.BlockSpec(shape, lambda i: (0, 0))
    row = lambda shape: pl.BlockSpec(shape, lambda i: (i, 0))
    return pl.pallas_call(
        _dense_body,
        grid=grid,
        in_specs=[
            row((_BLK, F)),
            row((_BLK, F)),
            full((F, H)),
            full((F, H)),
            full((H, C)),
            full((1, H)),
            full((1, C)),
        ],
        out_specs=[row((_BLK, C)), full((C, C))],
        out_shape=[
            jax.ShapeDtypeStruct((N, C), jnp.float32),
            jax.ShapeDtypeStruct((C, C), jnp.float32),
        ],
    )(x, agg, W1_rel.T, W1_root.T, W_mlp.T, b1_rel.reshape(1, H),
      b_mlp.reshape(1, C))


def _scalar_body(ssT_ref, nd_ref, mc_ref, o_ref):
    ss = ssT_ref[...]
    ssn = jnp.sqrt(jnp.sum(ss * ss))
    r = jax.lax.broadcasted_iota(jnp.int32, (C, C), 0)
    c = jax.lax.broadcasted_iota(jnp.int32, (C, C), 1)
    eye = jnp.where(r == c, 1.0, 0.0)
    diff = ss / ssn - eye / jnp.sqrt(jnp.float32(C))
    o_ref[...] = jnp.sqrt(jnp.sum(diff * diff)).reshape(1, 1)
    mc_ref[...] = (-(nd_ref[0, 0] / nd_ref[0, 1])).reshape(1, 1)


def _scalar_stage(ssT, num, den):
    nd = jnp.stack([num, den]).reshape(1, 2)
    mc, o = pl.pallas_call(
        _scalar_body,
        out_shape=[
            jax.ShapeDtypeStruct((1, 1), jnp.float32),
            jax.ShapeDtypeStruct((1, 1), jnp.float32),
        ],
    )(ssT, nd)
    return mc.reshape(()), o.reshape(())


def kernel(x, edge_index, edge_weight, W1_rel, b1_rel, W1_root, W_mlp, b_mlp):
    src = edge_index[0]
    dst = edge_index[1]

    msg = x[src] * edge_weight[:, None]
    agg = jax.ops.segment_sum(msg, dst, num_segments=N)

    s_soft, ssT = _dense_stage(x, agg, W1_rel, b1_rel, W1_root, W_mlp, b_mlp)

    p_src = s_soft[src]
    p_dst = s_soft[dst]
    num = jnp.sum(p_src * p_dst)
    den = jnp.sum(p_src * p_src)

    mc_loss, o_loss = _scalar_stage(ssT, num, den)

    adj = jnp.zeros((N, N), jnp.float32).at[src, dst].add(1.0)

    return (s_soft, mc_loss, o_loss, adj[None])


# speedup vs baseline: 1.3843x; 1.3843x over previous
import functools

import jax
import jax.numpy as jnp
from jax import lax
from jax.experimental import pallas as pl
from jax.experimental.pallas import tpu as pltpu
from jax.experimental.pallas import tpu_sc as plsc

N = 10000
E = 160000
F = 128
H = 128
C = 16

_NC = 2
_NS = 16
_NW = _NC * _NS
_EP = E // _NW
_K = 200
_T = _EP // _K
_NP = 10240
_NROWS = _NP // _NS

_sc_mesh = functools.partial(
    plsc.VectorSubcoreMesh, core_axis_name="c", subcore_axis_name="s")


def _agg_body(src_hbm, dst_hbm, w_hbm, x_hbm, zrow_hbm, out_hbm,
              src_v, dst_v, w_v, rows_v, agg_sp, sem):
    cid = lax.axis_index("c")
    sid = lax.axis_index("s")
    wid = sid * _NC + cid

    pltpu.sync_copy(zrow_hbm, agg_sp.at[pl.ds(sid * _NROWS, _NROWS)])
    plsc.subcore_barrier()

    def chunk(t, carry):
        base = pl.multiple_of(wid * _EP + t * _K, 8)
        pltpu.sync_copy(src_hbm.at[pl.ds(base, _K)], src_v)
        pltpu.sync_copy(dst_hbm.at[pl.ds(base, _K)], dst_v)
        pltpu.sync_copy(w_hbm.at[pl.ds(base, _K)], w_v)
        pltpu.async_copy(x_hbm.at[src_v], rows_v, sem).wait()

        def scale(r, carry2):
            wv = plsc.load_gather(w_v, [jnp.full((16,), r, jnp.int32)])
            for k in range(F // 16):
                sl = pl.ds(16 * k, 16)
                rows_v[r, sl] = rows_v[r, sl] * wv
            return carry2

        lax.fori_loop(0, _K, scale, 0)
        pltpu.sync_copy(rows_v, agg_sp.at[dst_v], add=True)
        return carry

    lax.fori_loop(0, _T, chunk, 0)

    plsc.subcore_barrier()
    pltpu.sync_copy(agg_sp.at[pl.ds(sid * _NROWS, _NROWS)],
                    out_hbm.at[cid, pl.ds(sid * _NROWS, _NROWS)])


def _agg_stage(src, dst, edge_weight, x, zrow):
    k = pl.kernel(
        _agg_body,
        out_type=jax.ShapeDtypeStruct((_NC, _NP, F), jnp.float32),
        mesh=_sc_mesh(),
        compiler_params=pltpu.CompilerParams(needs_layout_passes=False),
        scratch_types=[
            pltpu.VMEM((_K,), jnp.int32),
            pltpu.VMEM((_K,), jnp.int32),
            pltpu.VMEM((_K,), jnp.float32),
            pltpu.VMEM((_K, F), jnp.float32),
            pltpu.VMEM_SHARED((_NP, F), jnp.float32),
            pltpu.SemaphoreType.DMA,
        ],
    )
    return k(src, dst, edge_weight, x, zrow)

_BLK = 1000


def _dense_body(x_ref, agg0_ref, agg1_ref, wrelT_ref, wrootT_ref, wmlpT_ref,
                brel_ref, bmlp_ref, s_ref, ssT_ref):
    i = pl.program_id(0)
    agg = agg0_ref[...] + agg1_ref[...]
    h = jnp.dot(agg, wrelT_ref[...], preferred_element_type=jnp.float32)
    h = h + brel_ref[...]
    h = h + jnp.dot(x_ref[...], wrootT_ref[...], preferred_element_type=jnp.float32)
    h = jnp.maximum(h, 0.0)
    sl = jnp.dot(h, wmlpT_ref[...], preferred_element_type=jnp.float32) + bmlp_ref[...]
    m = jnp.max(sl, axis=-1, keepdims=True)
    e = jnp.exp(sl - m)
    p = e / jnp.sum(e, axis=-1, keepdims=True)
    s_ref[...] = p
    blk = jax.lax.dot_general(p, p, (((0,), (0,)), ((), ())),
                              preferred_element_type=jnp.float32)

    @pl.when(i == 0)
    def _():
        ssT_ref[...] = blk

    @pl.when(i != 0)
    def _():
        ssT_ref[...] += blk


def _dense_stage(x, agg0, agg1, W1_rel, b1_rel, W1_root, W_mlp, b_mlp):
    grid = (N // _BLK,)
    full = lambda shape: pl.BlockSpec(shape, lambda i: (0, 0))
    row = lambda shape: pl.BlockSpec(shape, lambda i: (i, 0))
    return pl.pallas_call(
        _dense_body,
        grid=grid,
        in_specs=[
            row((_BLK, F)),
            row((_BLK, F)),
            row((_BLK, F)),
            full((F, H)),
            full((F, H)),
            full((H, C)),
            full((1, H)),
            full((1, C)),
        ],
        out_specs=[row((_BLK, C)), full((C, C))],
        out_shape=[
            jax.ShapeDtypeStruct((N, C), jnp.float32),
            jax.ShapeDtypeStruct((C, C), jnp.float32),
        ],
    )(x, agg0, agg1, W1_rel.T, W1_root.T, W_mlp.T, b1_rel.reshape(1, H),
      b_mlp.reshape(1, C))


def _scalar_body(ssT_ref, nd_ref, mc_ref, o_ref):
    ss = ssT_ref[...]
    ssn = jnp.sqrt(jnp.sum(ss * ss))
    r = jax.lax.broadcasted_iota(jnp.int32, (C, C), 0)
    c = jax.lax.broadcasted_iota(jnp.int32, (C, C), 1)
    eye = jnp.where(r == c, 1.0, 0.0)
    diff = ss / ssn - eye / jnp.sqrt(jnp.float32(C))
    o_ref[...] = jnp.sqrt(jnp.sum(diff * diff)).reshape(1, 1)
    mc_ref[...] = (-(nd_ref[0, 0] / nd_ref[0, 1])).reshape(1, 1)


def _scalar_stage(ssT, num, den):
    nd = jnp.stack([num, den]).reshape(1, 2)
    mc, o = pl.pallas_call(
        _scalar_body,
        out_shape=[
            jax.ShapeDtypeStruct((1, 1), jnp.float32),
            jax.ShapeDtypeStruct((1, 1), jnp.float32),
        ],
    )(ssT, nd)
    return mc.reshape(()), o.reshape(())


def kernel(x, edge_index, edge_weight, W1_rel, b1_rel, W1_root, W_mlp, b_mlp):
    src = edge_index[0]
    dst = edge_index[1]

    zrow = jnp.zeros((_NROWS, F), jnp.float32)
    aggp = _agg_stage(src, dst, edge_weight, x, zrow)

    s_soft, ssT = _dense_stage(x, aggp[0, :N], aggp[1, :N], W1_rel, b1_rel,
                               W1_root, W_mlp, b_mlp)

    p_src = s_soft[src]
    p_dst = s_soft[dst]
    num = jnp.sum(p_src * p_dst)
    den = jnp.sum(p_src * p_src)

    mc_loss, o_loss = _scalar_stage(ssT, num, den)

    adj = jnp.zeros((N, N), jnp.float32).at[src, dst].add(1.0)

    return (s_soft, mc_loss, o_loss, adj[None])


# speedup vs baseline: 1.4371x; 1.0382x over previous
import functools

import jax
import jax.numpy as jnp
from jax import lax
from jax.experimental import pallas as pl
from jax.experimental.pallas import tpu as pltpu
from jax.experimental.pallas import tpu_sc as plsc

N = 10000
E = 160000
F = 128
H = 128
C = 16

_NC = 2
_NS = 16
_NW = _NC * _NS
_EP = E // _NW
_K = 200
_T = _EP // _K
_NP = 10240
_NROWS = _NP // _NS

_sc_mesh = functools.partial(
    plsc.VectorSubcoreMesh, core_axis_name="c", subcore_axis_name="s")


def _agg_body(src_hbm, dst_hbm, w_hbm, x_hbm, zrow_hbm, out_hbm,
              src_v, dst_v, w_v, rows_v, agg_sp, sem):
    cid = lax.axis_index("c")
    sid = lax.axis_index("s")
    wid = sid * _NC + cid

    pltpu.sync_copy(zrow_hbm, agg_sp.at[pl.ds(sid * _NROWS, _NROWS)])
    plsc.subcore_barrier()

    def chunk(t, carry):
        base = pl.multiple_of(wid * _EP + t * _K, 8)
        pltpu.sync_copy(src_hbm.at[pl.ds(base, _K)], src_v)
        pltpu.sync_copy(dst_hbm.at[pl.ds(base, _K)], dst_v)
        pltpu.sync_copy(w_hbm.at[pl.ds(base, _K)], w_v)
        pltpu.async_copy(x_hbm.at[src_v], rows_v, sem).wait()

        def scale(r, carry2):
            wv = plsc.load_gather(w_v, [jnp.full((16,), r, jnp.int32)])
            for k in range(F // 16):
                sl = pl.ds(16 * k, 16)
                rows_v[r, sl] = rows_v[r, sl] * wv
            return carry2

        lax.fori_loop(0, _K, scale, 0)
        pltpu.sync_copy(rows_v, agg_sp.at[dst_v], add=True)
        return carry

    lax.fori_loop(0, _T, chunk, 0)

    plsc.subcore_barrier()
    pltpu.sync_copy(agg_sp.at[pl.ds(sid * _NROWS, _NROWS)],
                    out_hbm.at[cid, pl.ds(sid * _NROWS, _NROWS)])


def _agg_stage(src, dst, edge_weight, x, zrow):
    k = pl.kernel(
        _agg_body,
        out_type=jax.ShapeDtypeStruct((_NC, _NP, F), jnp.float32),
        mesh=_sc_mesh(),
        compiler_params=pltpu.CompilerParams(needs_layout_passes=False),
        scratch_types=[
            pltpu.VMEM((_K,), jnp.int32),
            pltpu.VMEM((_K,), jnp.int32),
            pltpu.VMEM((_K,), jnp.float32),
            pltpu.VMEM((_K, F), jnp.float32),
            pltpu.VMEM_SHARED((_NP, F), jnp.float32),
            pltpu.SemaphoreType.DMA,
        ],
    )
    return k(src, dst, edge_weight, x, zrow)

_A = N * N
_ZBLK = 2_000_000


_ZR, _ZC, _ZB = 8000, 12500, 80


def _zero_body(out_ref):
    out_ref[...] = jnp.zeros((_ZB, _ZC), jnp.float32)


def _zero_stage():
    z = pl.pallas_call(
        _zero_body,
        grid=(_ZR // _ZB,),
        out_specs=pl.BlockSpec((_ZB, _ZC), lambda i: (i, 0)),
        out_shape=jax.ShapeDtypeStruct((_ZR, _ZC), jnp.float32),
    )()
    return z.reshape(_A)


_NWIN = -(-_K // 16)


def _woff(i):
    return jnp.minimum(16 * i, _K - 16)


def _edge_flat(src_v, dst_v, f_v):
    def step(i, carry):
        sl = pl.ds(_woff(i), 16)
        f_v[sl] = src_v[sl] * N + dst_v[sl]
        return carry
    lax.fori_loop(0, _NWIN, step, 0)


def _ids_body(src_hbm, dst_hbm, s_hbm, adj_ref, nd_out,
              src_v, dst_v, f_v, vals_v, ssrc_v, sdst_v, acc_v, sem):
    cid = lax.axis_index("c")
    sid = lax.axis_index("s")
    wid = sid * _NC + cid

    def chunk(t, carry):
        nacc, dacc = carry
        base = pl.multiple_of(wid * _EP + t * _K, 8)
        pltpu.sync_copy(src_hbm.at[pl.ds(base, _K)], src_v)
        pltpu.sync_copy(dst_hbm.at[pl.ds(base, _K)], dst_v)
        _edge_flat(src_v, dst_v, f_v)

        def wrt(i, carry2):
            off = _woff(i)
            sl = pl.ds(off, 16)
            eid = lax.iota(jnp.int32, 16) + (base + off)
            vals_v[sl] = eid.astype(jnp.float32)
            return carry2
        lax.fori_loop(0, _NWIN, wrt, 0)
        pltpu.sync_copy(vals_v, adj_ref.at[f_v])

        a = pltpu.async_copy(s_hbm.at[src_v], ssrc_v, sem)
        b = pltpu.async_copy(s_hbm.at[dst_v], sdst_v, sem)
        a.wait()
        b.wait()

        def red(r, carry2):
            na, da = carry2
            sa = ssrc_v[r, pl.ds(0, C)]
            sb = sdst_v[r, pl.ds(0, C)]
            return na + sa * sb, da + sa * sa
        nacc, dacc = lax.fori_loop(0, _K, red, (nacc, dacc))
        return nacc, dacc

    z = jnp.zeros((16,), jnp.float32)
    nacc, dacc = lax.fori_loop(0, _T, chunk, (z, z))
    acc_v[pl.ds(0, 16)] = nacc
    acc_v[pl.ds(16, 16)] = dacc
    pltpu.sync_copy(acc_v, nd_out.at[wid, 0])


def _ids_stage(src, dst, s_soft, adj_ref):
    k = pl.kernel(
        _ids_body,
        out_type=jax.ShapeDtypeStruct((_NW, 1, 32), jnp.float32),
        mesh=_sc_mesh(),
        compiler_params=pltpu.CompilerParams(needs_layout_passes=False),
        scratch_types=[
            pltpu.VMEM((_K,), jnp.int32),
            pltpu.VMEM((_K,), jnp.int32),
            pltpu.VMEM((_K,), jnp.int32),
            pltpu.VMEM((_K,), jnp.float32),
            pltpu.VMEM((_K, F), jnp.float32),
            pltpu.VMEM((_K, F), jnp.float32),
            pltpu.VMEM((32,), jnp.float32),
            pltpu.SemaphoreType.DMA,
        ],
    )
    return k(src, dst, s_soft, adj_ref)


_CNT_STRIPE = E // _NS


def _cnt_body(src_hbm, dst_hbm, adj_ref, cnt_out, g_out,
              src_v, dst_v, f_v, g_v, gi_v, ones_v, zv, cnt_sp, sem):
    cid = lax.axis_index("c")
    sid = lax.axis_index("s")
    wid = sid * _NC + cid

    def zfill(i, carry):
        zv[pl.ds(16 * i, 16)] = jnp.zeros((16,), jnp.float32)
        return carry
    lax.fori_loop(0, 2000 // 16, zfill, 0)

    def zcp(j, carry):
        pltpu.sync_copy(zv, cnt_sp.at[pl.ds(sid * _CNT_STRIPE + j * 2000, 2000)])
        return carry
    lax.fori_loop(0, _CNT_STRIPE // 2000, zcp, 0)

    def one(i, carry):
        ones_v[pl.ds(_woff(i), 16)] = jnp.full((16,), 1.0, jnp.float32)
        return carry
    lax.fori_loop(0, _NWIN, one, 0)
    plsc.subcore_barrier()

    def chunk(t, carry):
        base = pl.multiple_of(wid * _EP + t * _K, 8)
        pltpu.sync_copy(src_hbm.at[pl.ds(base, _K)], src_v)
        pltpu.sync_copy(dst_hbm.at[pl.ds(base, _K)], dst_v)
        _edge_flat(src_v, dst_v, f_v)
        pltpu.async_copy(adj_ref.at[f_v], g_v, sem).wait()

        def conv(i, carry2):
            sl = pl.ds(_woff(i), 16)
            gi_v[sl] = g_v[sl].astype(jnp.int32)
            return carry2
        lax.fori_loop(0, _NWIN, conv, 0)
        pltpu.sync_copy(ones_v, cnt_sp.at[gi_v], add=True)
        pltpu.sync_copy(gi_v, g_out.at[pl.ds(base, _K)])
        return carry

    lax.fori_loop(0, _T, chunk, 0)
    plsc.subcore_barrier()

    def dump(j, carry):
        pltpu.sync_copy(cnt_sp.at[pl.ds(sid * _CNT_STRIPE + j * 2000, 2000)], zv)
        pltpu.sync_copy(zv, cnt_out.at[cid, sid, j, 0])
        return carry
    lax.fori_loop(0, _CNT_STRIPE // 2000, dump, 0)


def _cnt_stage(src, dst, adj_ref):
    k = pl.kernel(
        _cnt_body,
        out_type=[
            jax.ShapeDtypeStruct((_NC, _NS, _CNT_STRIPE // 2000, 1, 2000), jnp.float32),
            jax.ShapeDtypeStruct((E,), jnp.int32),
        ],
        mesh=_sc_mesh(),
        compiler_params=pltpu.CompilerParams(needs_layout_passes=False),
        scratch_types=[
            pltpu.VMEM((_K,), jnp.int32),
            pltpu.VMEM((_K,), jnp.int32),
            pltpu.VMEM((_K,), jnp.int32),
            pltpu.VMEM((_K,), jnp.float32),
            pltpu.VMEM((_K,), jnp.int32),
            pltpu.VMEM((_K,), jnp.float32),
            pltpu.VMEM((2000,), jnp.float32),
            pltpu.VMEM_SHARED((E,), jnp.float32),
            pltpu.SemaphoreType.DMA,
        ],
    )
    return k(src, dst, adj_ref)


def _fin_body(src_hbm, dst_hbm, g_hbm, c0_hbm, c1_hbm, adj_ref,
              src_v, dst_v, f_v, gi_v, c0_v, c1_v, vals_v, sem):
    cid = lax.axis_index("c")
    sid = lax.axis_index("s")
    wid = sid * _NC + cid

    def chunk(t, carry):
        base = pl.multiple_of(wid * _EP + t * _K, 8)
        pltpu.sync_copy(src_hbm.at[pl.ds(base, _K)], src_v)
        pltpu.sync_copy(dst_hbm.at[pl.ds(base, _K)], dst_v)
        _edge_flat(src_v, dst_v, f_v)
        pltpu.sync_copy(g_hbm.at[pl.ds(base, _K)], gi_v)
        a = pltpu.async_copy(c0_hbm.at[gi_v], c0_v, sem)
        b = pltpu.async_copy(c1_hbm.at[gi_v], c1_v, sem)
        a.wait()
        b.wait()

        def add(i, carry2):
            sl = pl.ds(_woff(i), 16)
            vals_v[sl] = c0_v[sl] + c1_v[sl]
            return carry2
        lax.fori_loop(0, _NWIN, add, 0)
        pltpu.sync_copy(vals_v, adj_ref.at[f_v])
        return carry

    lax.fori_loop(0, _T, chunk, 0)


def _fin_stage(src, dst, gids, cnt0, cnt1, adj_ref):
    k = pl.kernel(
        _fin_body,
        out_type=(),
        mesh=_sc_mesh(),
        compiler_params=pltpu.CompilerParams(needs_layout_passes=False),
        scratch_types=[
            pltpu.VMEM((_K,), jnp.int32),
            pltpu.VMEM((_K,), jnp.int32),
            pltpu.VMEM((_K,), jnp.int32),
            pltpu.VMEM((_K,), jnp.int32),
            pltpu.VMEM((_K,), jnp.float32),
            pltpu.VMEM((_K,), jnp.float32),
            pltpu.VMEM((_K,), jnp.float32),
            pltpu.SemaphoreType.DMA,
        ],
    )
    return k(src, dst, gids, cnt0, cnt1, adj_ref)


_BLK = 1000


def _dense_body(x_ref, agg0_ref, agg1_ref, wrelT_ref, wrootT_ref, wmlpT_ref,
                brel_ref, bmlp_ref, s_ref, spad_ref, ssT_ref):
    i = pl.program_id(0)
    agg = agg0_ref[...] + agg1_ref[...]
    h = jnp.dot(agg, wrelT_ref[...], preferred_element_type=jnp.float32)
    h = h + brel_ref[...]
    h = h + jnp.dot(x_ref[...], wrootT_ref[...], preferred_element_type=jnp.float32)
    h = jnp.maximum(h, 0.0)
    sl = jnp.dot(h, wmlpT_ref[...], preferred_element_type=jnp.float32) + bmlp_ref[...]
    m = jnp.max(sl, axis=-1, keepdims=True)
    e = jnp.exp(sl - m)
    p = e / jnp.sum(e, axis=-1, keepdims=True)
    s_ref[...] = p
    spad_ref[...] = jnp.concatenate(
        [p, jnp.zeros((_BLK, F - C), jnp.float32)], axis=1)
    blk = jax.lax.dot_general(p, p, (((0,), (0,)), ((), ())),
                              preferred_element_type=jnp.float32)

    @pl.when(i == 0)
    def _():
        ssT_ref[...] = blk

    @pl.when(i != 0)
    def _():
        ssT_ref[...] += blk


def _dense_stage(x, agg0, agg1, W1_rel, b1_rel, W1_root, W_mlp, b_mlp):
    grid = (N // _BLK,)
    full = lambda shape: pl.BlockSpec(shape, lambda i: (0, 0))
    row = lambda shape: pl.BlockSpec(shape, lambda i: (i, 0))
    return pl.pallas_call(
        _dense_body,
        grid=grid,
        in_specs=[
            row((_BLK, F)),
            row((_BLK, F)),
            row((_BLK, F)),
            full((F, H)),
            full((F, H)),
            full((H, C)),
            full((1, H)),
            full((1, C)),
        ],
        out_specs=[row((_BLK, C)), row((_BLK, F)), full((C, C))],
        out_shape=[
            jax.ShapeDtypeStruct((N, C), jnp.float32),
            jax.ShapeDtypeStruct((N, F), jnp.float32),
            jax.ShapeDtypeStruct((C, C), jnp.float32),
        ],
    )(x, agg0, agg1, W1_rel.T, W1_root.T, W_mlp.T, b1_rel.reshape(1, H),
      b_mlp.reshape(1, C))


def _scalar_body(ssT_ref, nd_ref, mc_ref, o_ref):
    ss = ssT_ref[...]
    ssn = jnp.sqrt(jnp.sum(ss * ss))
    r = jax.lax.broadcasted_iota(jnp.int32, (C, C), 0)
    c = jax.lax.broadcasted_iota(jnp.int32, (C, C), 1)
    eye = jnp.where(r == c, 1.0, 0.0)
    diff = ss / ssn - eye / jnp.sqrt(jnp.float32(C))
    o_ref[...] = jnp.sqrt(jnp.sum(diff * diff)).reshape(1, 1)
    nd = nd_ref[...]
    num = jnp.sum(nd[:, :16])
    den = jnp.sum(nd[:, 16:])
    mc_ref[...] = (-(num / den)).reshape(1, 1)


def _scalar_stage(ssT, nd):
    mc, o = pl.pallas_call(
        _scalar_body,
        out_shape=[
            jax.ShapeDtypeStruct((1, 1), jnp.float32),
            jax.ShapeDtypeStruct((1, 1), jnp.float32),
        ],
    )(ssT, nd.reshape(_NW, 32))
    return mc.reshape(()), o.reshape(())


def kernel(x, edge_index, edge_weight, W1_rel, b1_rel, W1_root, W_mlp, b_mlp):
    src = edge_index[0]
    dst = edge_index[1]

    zrow = jnp.zeros((_NROWS, F), jnp.float32)
    aggp = _agg_stage(src, dst, edge_weight, x, zrow)

    s_soft, s_pad, ssT = _dense_stage(x, aggp[0, :N], aggp[1, :N], W1_rel,
                                      b1_rel, W1_root, W_mlp, b_mlp)

    adj_ref = jax.new_ref(_zero_stage())
    nd = _ids_stage(src, dst, s_pad, adj_ref)
    cnt, gids = _cnt_stage(src, dst, adj_ref)
    cnt = cnt.reshape(_NC, E)
    _fin_stage(src, dst, gids, cnt[0], cnt[1], adj_ref)
    adj = adj_ref[...]

    mc_loss, o_loss = _scalar_stage(ssT, nd)

    return (s_soft, mc_loss, o_loss, adj.reshape(1, N, N))


# speedup vs baseline: 1.4376x; 1.0004x over previous
import functools

import jax
import jax.numpy as jnp
from jax import lax
from jax.experimental import pallas as pl
from jax.experimental.pallas import tpu as pltpu
from jax.experimental.pallas import tpu_sc as plsc

N = 10000
E = 160000
F = 128
H = 128
C = 16

_NC = 2
_NS = 16
_NW = _NC * _NS
_EP = E // _NW
_K = 200
_T = _EP // _K
_NP = 10240
_NROWS = _NP // _NS

_sc_mesh = functools.partial(
    plsc.VectorSubcoreMesh, core_axis_name="c", subcore_axis_name="s")


def _agg_body(src_hbm, dst_hbm, w_hbm, x_hbm, zrow_hbm, out_hbm,
              src_v, dst_v, w_v, rows_v, agg_sp, sem):
    cid = lax.axis_index("c")
    sid = lax.axis_index("s")
    wid = sid * _NC + cid

    pltpu.sync_copy(zrow_hbm, agg_sp.at[pl.ds(sid * _NROWS, _NROWS)])
    plsc.subcore_barrier()

    def chunk(t, carry):
        base = pl.multiple_of(wid * _EP + t * _K, 8)
        pltpu.sync_copy(src_hbm.at[pl.ds(base, _K)], src_v)
        pltpu.sync_copy(dst_hbm.at[pl.ds(base, _K)], dst_v)
        pltpu.sync_copy(w_hbm.at[pl.ds(base, _K)], w_v)
        pltpu.async_copy(x_hbm.at[src_v], rows_v, sem).wait()

        def scale(r, carry2):
            wv = plsc.load_gather(w_v, [jnp.full((16,), r, jnp.int32)])
            for k in range(F // 16):
                sl = pl.ds(16 * k, 16)
                rows_v[r, sl] = rows_v[r, sl] * wv
            return carry2

        lax.fori_loop(0, _K, scale, 0)
        pltpu.sync_copy(rows_v, agg_sp.at[dst_v], add=True)
        return carry

    lax.fori_loop(0, _T, chunk, 0)

    plsc.subcore_barrier()
    pltpu.sync_copy(agg_sp.at[pl.ds(sid * _NROWS, _NROWS)],
                    out_hbm.at[cid, pl.ds(sid * _NROWS, _NROWS)])


def _agg_stage(src, dst, edge_weight, x, zrow):
    k = pl.kernel(
        _agg_body,
        out_type=jax.ShapeDtypeStruct((_NC, _NP, F), jnp.float32),
        mesh=_sc_mesh(),
        compiler_params=pltpu.CompilerParams(needs_layout_passes=False),
        scratch_types=[
            pltpu.VMEM((_K,), jnp.int32),
            pltpu.VMEM((_K,), jnp.int32),
            pltpu.VMEM((_K,), jnp.float32),
            pltpu.VMEM((_K, F), jnp.float32),
            pltpu.VMEM_SHARED((_NP, F), jnp.float32),
            pltpu.SemaphoreType.DMA,
        ],
    )
    return k(src, dst, edge_weight, x, zrow)

_A = N * N
_ZBLK = 2_000_000


_ZR, _ZC, _ZB = 8000, 12500, 80


def _zero_body(out_ref):
    out_ref[...] = jnp.zeros((_ZB, _ZC), jnp.float32)


def _zero_stage():
    z = pl.pallas_call(
        _zero_body,
        grid=(_ZR // _ZB,),
        out_specs=pl.BlockSpec((_ZB, _ZC), lambda i: (i, 0)),
        out_shape=jax.ShapeDtypeStruct((_ZR, _ZC), jnp.float32),
    )()
    return z.reshape(_A)


_NWIN = -(-_K // 16)


def _woff(i):
    return jnp.minimum(16 * i, _K - 16)


def _edge_flat(src_v, dst_v, f_v):
    def step(i, carry):
        sl = pl.ds(_woff(i), 16)
        f_v[sl] = src_v[sl] * N + dst_v[sl]
        return carry
    lax.fori_loop(0, _NWIN, step, 0)


def _ids_body(src_hbm, dst_hbm, s_hbm, adj_ref, nd_out,
              src_v, dst_v, f_v, vals_v, ssrc_v, sdst_v, acc_v, sem):
    cid = lax.axis_index("c")
    sid = lax.axis_index("s")
    wid = sid * _NC + cid

    def chunk(t, carry):
        nacc, dacc = carry
        base = pl.multiple_of(wid * _EP + t * _K, 8)
        pltpu.sync_copy(src_hbm.at[pl.ds(base, _K)], src_v)
        pltpu.sync_copy(dst_hbm.at[pl.ds(base, _K)], dst_v)
        _edge_flat(src_v, dst_v, f_v)

        def wrt(i, carry2):
            off = _woff(i)
            sl = pl.ds(off, 16)
            eid = lax.iota(jnp.int32, 16) + (base + off)
            vals_v[sl] = eid.astype(jnp.float32)
            return carry2
        lax.fori_loop(0, _NWIN, wrt, 0)
        pltpu.sync_copy(vals_v, adj_ref.at[f_v])

        a = pltpu.async_copy(s_hbm.at[src_v], ssrc_v, sem)
        b = pltpu.async_copy(s_hbm.at[dst_v], sdst_v, sem)
        a.wait()
        b.wait()

        def red(r, carry2):
            na, da = carry2
            sa = ssrc_v[r, pl.ds(0, C)]
            sb = sdst_v[r, pl.ds(0, C)]
            return na + sa * sb, da + sa * sa
        nacc, dacc = lax.fori_loop(0, _K, red, (nacc, dacc))
        return nacc, dacc

    z = jnp.zeros((16,), jnp.float32)
    nacc, dacc = lax.fori_loop(0, _T, chunk, (z, z))
    acc_v[pl.ds(0, 16)] = nacc
    acc_v[pl.ds(16, 16)] = dacc
    pltpu.sync_copy(acc_v, nd_out.at[wid, 0])


def _ids_stage(src, dst, s_soft, adj_ref):
    k = pl.kernel(
        _ids_body,
        out_type=jax.ShapeDtypeStruct((_NW, 1, 32), jnp.float32),
        mesh=_sc_mesh(),
        compiler_params=pltpu.CompilerParams(needs_layout_passes=False),
        scratch_types=[
            pltpu.VMEM((_K,), jnp.int32),
            pltpu.VMEM((_K,), jnp.int32),
            pltpu.VMEM((_K,), jnp.int32),
            pltpu.VMEM((_K,), jnp.float32),
            pltpu.VMEM((_K, F), jnp.float32),
            pltpu.VMEM((_K, F), jnp.float32),
            pltpu.VMEM((32,), jnp.float32),
            pltpu.SemaphoreType.DMA,
        ],
    )
    return k(src, dst, s_soft, adj_ref)


_CNT_STRIPE = E // _NS


def _cnt_body(src_hbm, dst_hbm, adj_ref, cnt_out, g_out,
              src_v, dst_v, f_v, g_v, gi_v, ones_v, zv, cnt_sp, sem):
    cid = lax.axis_index("c")
    sid = lax.axis_index("s")
    wid = sid * _NC + cid

    def zfill(i, carry):
        zv[pl.ds(16 * i, 16)] = jnp.zeros((16,), jnp.float32)
        return carry
    lax.fori_loop(0, 2000 // 16, zfill, 0)

    def zcp(j, carry):
        pltpu.sync_copy(zv, cnt_sp.at[pl.ds(sid * _CNT_STRIPE + j * 2000, 2000)])
        return carry
    lax.fori_loop(0, _CNT_STRIPE // 2000, zcp, 0)

    def one(i, carry):
        ones_v[pl.ds(_woff(i), 16)] = jnp.full((16,), 1.0, jnp.float32)
        return carry
    lax.fori_loop(0, _NWIN, one, 0)
    plsc.subcore_barrier()

    def chunk(t, carry):
        base = pl.multiple_of(wid * _EP + t * _K, 8)
        pltpu.sync_copy(src_hbm.at[pl.ds(base, _K)], src_v)
        pltpu.sync_copy(dst_hbm.at[pl.ds(base, _K)], dst_v)
        _edge_flat(src_v, dst_v, f_v)
        pltpu.async_copy(adj_ref.at[f_v], g_v, sem).wait()

        def conv(i, carry2):
            sl = pl.ds(_woff(i), 16)
            gi_v[sl] = g_v[sl].astype(jnp.int32)
            return carry2
        lax.fori_loop(0, _NWIN, conv, 0)
        pltpu.sync_copy(ones_v, cnt_sp.at[gi_v], add=True)
        pltpu.sync_copy(gi_v, g_out.at[pl.ds(base, _K)])
        return carry

    lax.fori_loop(0, _T, chunk, 0)
    plsc.subcore_barrier()

    def dump(j, carry):
        pltpu.sync_copy(cnt_sp.at[pl.ds(sid * _CNT_STRIPE + j * 2000, 2000)], zv)
        pltpu.sync_copy(zv, cnt_out.at[cid, sid, j, 0])
        return carry
    lax.fori_loop(0, _CNT_STRIPE // 2000, dump, 0)


def _cnt_stage(src, dst, adj_ref):
    k = pl.kernel(
        _cnt_body,
        out_type=[
            jax.ShapeDtypeStruct((_NC, _NS, _CNT_STRIPE // 2000, 1, 2000), jnp.float32),
            jax.ShapeDtypeStruct((E,), jnp.int32),
        ],
        mesh=_sc_mesh(),
        compiler_params=pltpu.CompilerParams(needs_layout_passes=False),
        scratch_types=[
            pltpu.VMEM((_K,), jnp.int32),
            pltpu.VMEM((_K,), jnp.int32),
            pltpu.VMEM((_K,), jnp.int32),
            pltpu.VMEM((_K,), jnp.float32),
            pltpu.VMEM((_K,), jnp.int32),
            pltpu.VMEM((_K,), jnp.float32),
            pltpu.VMEM((2000,), jnp.float32),
            pltpu.VMEM_SHARED((E,), jnp.float32),
            pltpu.SemaphoreType.DMA,
        ],
    )
    return k(src, dst, adj_ref)


def _fin_body(src_hbm, dst_hbm, g_hbm, c0_hbm, c1_hbm, adj_ref,
              src_v, dst_v, f_v, gi_v, c0_v, c1_v, vals_v, sem):
    cid = lax.axis_index("c")
    sid = lax.axis_index("s")
    wid = sid * _NC + cid

    def chunk(t, carry):
        base = pl.multiple_of(wid * _EP + t * _K, 8)
        pltpu.sync_copy(src_hbm.at[pl.ds(base, _K)], src_v)
        pltpu.sync_copy(dst_hbm.at[pl.ds(base, _K)], dst_v)
        _edge_flat(src_v, dst_v, f_v)
        pltpu.sync_copy(g_hbm.at[pl.ds(base, _K)], gi_v)
        a = pltpu.async_copy(c0_hbm.at[gi_v], c0_v, sem)
        b = pltpu.async_copy(c1_hbm.at[gi_v], c1_v, sem)
        a.wait()
        b.wait()

        def add(i, carry2):
            sl = pl.ds(_woff(i), 16)
            vals_v[sl] = c0_v[sl] + c1_v[sl]
            return carry2
        lax.fori_loop(0, _NWIN, add, 0)
        pltpu.sync_copy(vals_v, adj_ref.at[f_v])
        return carry

    lax.fori_loop(0, _T, chunk, 0)


def _fin_stage(src, dst, gids, cnt0, cnt1, adj_ref):
    k = pl.kernel(
        _fin_body,
        out_type=(),
        mesh=_sc_mesh(),
        compiler_params=pltpu.CompilerParams(needs_layout_passes=False),
        scratch_types=[
            pltpu.VMEM((_K,), jnp.int32),
            pltpu.VMEM((_K,), jnp.int32),
            pltpu.VMEM((_K,), jnp.int32),
            pltpu.VMEM((_K,), jnp.int32),
            pltpu.VMEM((_K,), jnp.float32),
            pltpu.VMEM((_K,), jnp.float32),
            pltpu.VMEM((_K,), jnp.float32),
            pltpu.SemaphoreType.DMA,
        ],
    )
    return k(src, dst, gids, cnt0, cnt1, adj_ref)


_BLK = 1000


def _dense_body(x_ref, agg0_ref, agg1_ref, wrelT_ref, wrootT_ref, wmlpT_ref,
                brel_ref, bmlp_ref, s_ref, spad_ref, ssT_ref):
    i = pl.program_id(0)
    agg = agg0_ref[...] + agg1_ref[...]
    h = jnp.dot(agg, wrelT_ref[...], preferred_element_type=jnp.float32)
    h = h + brel_ref[...]
    h = h + jnp.dot(x_ref[...], wrootT_ref[...], preferred_element_type=jnp.float32)
    h = jnp.maximum(h, 0.0)
    sl = jnp.dot(h, wmlpT_ref[...], preferred_element_type=jnp.float32) + bmlp_ref[...]
    m = jnp.max(sl, axis=-1, keepdims=True)
    e = jnp.exp(sl - m)
    p = e / jnp.sum(e, axis=-1, keepdims=True)
    s_ref[...] = p
    spad_ref[...] = jnp.concatenate(
        [p, jnp.zeros((_BLK, F - C), jnp.float32)], axis=1)
    blk = jax.lax.dot_general(p, p, (((0,), (0,)), ((), ())),
                              preferred_element_type=jnp.float32)

    @pl.when(i == 0)
    def _():
        ssT_ref[...] = blk

    @pl.when(i != 0)
    def _():
        ssT_ref[...] += blk


def _dense_stage(x, agg0, agg1, W1_rel, b1_rel, W1_root, W_mlp, b_mlp):
    grid = (N // _BLK,)
    full = lambda shape: pl.BlockSpec(shape, lambda i: (0, 0))
    row = lambda shape: pl.BlockSpec(shape, lambda i: (i, 0))
    return pl.pallas_call(
        _dense_body,
        grid=grid,
        in_specs=[
            row((_BLK, F)),
            row((_BLK, F)),
            row((_BLK, F)),
            full((F, H)),
            full((F, H)),
            full((H, C)),
            full((1, H)),
            full((1, C)),
        ],
        out_specs=[row((_BLK, C)), row((_BLK, F)), full((C, C))],
        out_shape=[
            jax.ShapeDtypeStruct((N, C), jnp.float32),
            jax.ShapeDtypeStruct((N, F), jnp.float32),
            jax.ShapeDtypeStruct((C, C), jnp.float32),
        ],
    )(x, agg0, agg1, W1_rel.T, W1_root.T, W_mlp.T, b1_rel.reshape(1, H),
      b_mlp.reshape(1, C))


def _scalar_body(ssT_ref, nd_ref, mc_ref, o_ref):
    ss = ssT_ref[...]
    ssn = jnp.sqrt(jnp.sum(ss * ss))
    r = jax.lax.broadcasted_iota(jnp.int32, (C, C), 0)
    c = jax.lax.broadcasted_iota(jnp.int32, (C, C), 1)
    eye = jnp.where(r == c, 1.0, 0.0)
    diff = ss / ssn - eye / jnp.sqrt(jnp.float32(C))
    o_ref[...] = jnp.sqrt(jnp.sum(diff * diff)).reshape(1, 1)
    nd = nd_ref[...]
    num = jnp.sum(nd[:, :16])
    den = jnp.sum(nd[:, 16:])
    mc_ref[...] = (-(num / den)).reshape(1, 1)


def _scalar_stage(ssT, nd):
    mc, o = pl.pallas_call(
        _scalar_body,
        out_shape=[
            jax.ShapeDtypeStruct((1, 1), jnp.float32),
            jax.ShapeDtypeStruct((1, 1), jnp.float32),
        ],
    )(ssT, nd.reshape(_NW, 32))
    return mc.reshape(()), o.reshape(())


def kernel(x, edge_index, edge_weight, W1_rel, b1_rel, W1_root, W_mlp, b_mlp):
    src = edge_index[0]
    dst = edge_index[1]

    zrow = jnp.zeros((_NROWS, F), jnp.float32)
    aggp = _agg_stage(src, dst, edge_weight, x, zrow)

    s_soft, s_pad, ssT = _dense_stage(x, aggp[0, :N], aggp[1, :N], W1_rel,
                                      b1_rel, W1_root, W_mlp, b_mlp)

    adj_ref = jax.new_ref(_zero_stage())
    nd = _ids_stage(src, dst, s_pad, adj_ref)
    cnt, gids = _cnt_stage(src, dst, adj_ref)
    cnt = cnt.reshape(_NC, E)
    _fin_stage(src, dst, gids, cnt[0], cnt[1], adj_ref)
    adj = jax.freeze(adj_ref)

    mc_loss, o_loss = _scalar_stage(ssT, nd)

    return (s_soft, mc_loss, o_loss, adj.reshape(1, N, N))


# speedup vs baseline: 2.1912x; 1.5242x over previous
import functools

import jax
import jax.numpy as jnp
from jax import lax
from jax.experimental import pallas as pl
from jax.experimental.pallas import tpu as pltpu
from jax.experimental.pallas import tpu_sc as plsc

N = 10000
E = 160000
F = 128
H = 128
C = 16

_NC = 2
_NS = 16
_NW = _NC * _NS
_EP = E // _NW
_K = 200
_T = _EP // _K
_NP = 10240
_NROWS = _NP // _NS

_sc_mesh = functools.partial(
    plsc.VectorSubcoreMesh, core_axis_name="c", subcore_axis_name="s")


def _agg_body(src_hbm, dst_hbm, w_hbm, x_hbm, zrow_hbm, out_hbm,
              src_v, dst_v, w_v, rows_v, agg_sp, sem):
    cid = lax.axis_index("c")
    sid = lax.axis_index("s")
    wid = sid * _NC + cid

    pltpu.sync_copy(zrow_hbm, agg_sp.at[pl.ds(sid * _NROWS, _NROWS)])
    plsc.subcore_barrier()

    def chunk(t, carry):
        base = pl.multiple_of(wid * _EP + t * _K, 8)
        pltpu.sync_copy(src_hbm.at[pl.ds(base, _K)], src_v)
        pltpu.sync_copy(dst_hbm.at[pl.ds(base, _K)], dst_v)
        pltpu.sync_copy(w_hbm.at[pl.ds(base, _K)], w_v)
        pltpu.async_copy(x_hbm.at[src_v], rows_v, sem).wait()

        def scale(r, carry2):
            wv = plsc.load_gather(w_v, [jnp.full((16,), r, jnp.int32)])
            for k in range(F // 16):
                sl = pl.ds(16 * k, 16)
                rows_v[r, sl] = rows_v[r, sl] * wv
            return carry2

        lax.fori_loop(0, _K, scale, 0)
        pltpu.sync_copy(rows_v, agg_sp.at[dst_v], add=True)
        return carry

    lax.fori_loop(0, _T, chunk, 0)

    plsc.subcore_barrier()
    pltpu.sync_copy(agg_sp.at[pl.ds(sid * _NROWS, _NROWS)],
                    out_hbm.at[cid, pl.ds(sid * _NROWS, _NROWS)])


def _agg_stage(src, dst, edge_weight, x, zrow):
    k = pl.kernel(
        _agg_body,
        out_type=jax.ShapeDtypeStruct((_NC, _NP, F), jnp.float32),
        mesh=_sc_mesh(),
        compiler_params=pltpu.CompilerParams(needs_layout_passes=False),
        scratch_types=[
            pltpu.VMEM((_K,), jnp.int32),
            pltpu.VMEM((_K,), jnp.int32),
            pltpu.VMEM((_K,), jnp.float32),
            pltpu.VMEM((_K, F), jnp.float32),
            pltpu.VMEM_SHARED((_NP, F), jnp.float32),
            pltpu.SemaphoreType.DMA,
        ],
    )
    return k(src, dst, edge_weight, x, zrow)

_A = N * N
_ZBLK = 2_000_000


_ZBLK = 1 << 22


def _zero_body(out_ref):
    out_ref[...] = jnp.zeros((_ZBLK,), jnp.float32)


def _zero_stage():
    return pl.pallas_call(
        _zero_body,
        grid=(-(-_A // _ZBLK),),
        out_specs=pl.BlockSpec((_ZBLK,), lambda i: (i,)),
        out_shape=jax.ShapeDtypeStruct((_A,), jnp.float32),
    )()


_NWIN = -(-_K // 16)


def _woff(i):
    return jnp.minimum(16 * i, _K - 16)


def _edge_flat(src_v, dst_v, f_v):
    def step(i, carry):
        sl = pl.ds(_woff(i), 16)
        f_v[sl] = src_v[sl] * N + dst_v[sl]
        return carry
    lax.fori_loop(0, _NWIN, step, 0)


def _ids_body(src_hbm, dst_hbm, s_hbm, adj_ref, nd_out,
              src_v, dst_v, f_v, vals_v, ssrc_v, sdst_v, acc_v, sem):
    cid = lax.axis_index("c")
    sid = lax.axis_index("s")
    wid = sid * _NC + cid

    def chunk(t, carry):
        nacc, dacc = carry
        base = pl.multiple_of(wid * _EP + t * _K, 8)
        pltpu.sync_copy(src_hbm.at[pl.ds(base, _K)], src_v)
        pltpu.sync_copy(dst_hbm.at[pl.ds(base, _K)], dst_v)
        _edge_flat(src_v, dst_v, f_v)

        def wrt(i, carry2):
            off = _woff(i)
            sl = pl.ds(off, 16)
            eid = lax.iota(jnp.int32, 16) + (base + off)
            vals_v[sl] = eid.astype(jnp.float32)
            return carry2
        lax.fori_loop(0, _NWIN, wrt, 0)
        pltpu.sync_copy(vals_v, adj_ref.at[f_v])

        a = pltpu.async_copy(s_hbm.at[src_v], ssrc_v, sem)
        b = pltpu.async_copy(s_hbm.at[dst_v], sdst_v, sem)
        a.wait()
        b.wait()

        def red(r, carry2):
            na, da = carry2
            sa = ssrc_v[r, pl.ds(0, C)]
            sb = sdst_v[r, pl.ds(0, C)]
            return na + sa * sb, da + sa * sa
        nacc, dacc = lax.fori_loop(0, _K, red, (nacc, dacc))
        return nacc, dacc

    z = jnp.zeros((16,), jnp.float32)
    nacc, dacc = lax.fori_loop(0, _T, chunk, (z, z))
    acc_v[pl.ds(0, 16)] = nacc
    acc_v[pl.ds(16, 16)] = dacc
    pltpu.sync_copy(acc_v, nd_out.at[wid, 0])


def _ids_stage(src, dst, s_soft, adj_ref):
    k = pl.kernel(
        _ids_body,
        out_type=jax.ShapeDtypeStruct((_NW, 1, 32), jnp.float32),
        mesh=_sc_mesh(),
        compiler_params=pltpu.CompilerParams(needs_layout_passes=False),
        scratch_types=[
            pltpu.VMEM((_K,), jnp.int32),
            pltpu.VMEM((_K,), jnp.int32),
            pltpu.VMEM((_K,), jnp.int32),
            pltpu.VMEM((_K,), jnp.float32),
            pltpu.VMEM((_K, F), jnp.float32),
            pltpu.VMEM((_K, F), jnp.float32),
            pltpu.VMEM((32,), jnp.float32),
            pltpu.SemaphoreType.DMA,
        ],
    )
    return k(src, dst, s_soft, adj_ref)


_CNT_STRIPE = E // _NS


def _cnt_body(src_hbm, dst_hbm, adj_ref, cnt_out, g_out,
              src_v, dst_v, f_v, g_v, gi_v, ones_v, zv, cnt_sp, sem):
    cid = lax.axis_index("c")
    sid = lax.axis_index("s")
    wid = sid * _NC + cid

    def zfill(i, carry):
        zv[pl.ds(16 * i, 16)] = jnp.zeros((16,), jnp.float32)
        return carry
    lax.fori_loop(0, 2000 // 16, zfill, 0)

    def zcp(j, carry):
        pltpu.sync_copy(zv, cnt_sp.at[pl.ds(sid * _CNT_STRIPE + j * 2000, 2000)])
        return carry
    lax.fori_loop(0, _CNT_STRIPE // 2000, zcp, 0)

    def one(i, carry):
        ones_v[pl.ds(_woff(i), 16)] = jnp.full((16,), 1.0, jnp.float32)
        return carry
    lax.fori_loop(0, _NWIN, one, 0)
    plsc.subcore_barrier()

    def chunk(t, carry):
        base = pl.multiple_of(wid * _EP + t * _K, 8)
        pltpu.sync_copy(src_hbm.at[pl.ds(base, _K)], src_v)
        pltpu.sync_copy(dst_hbm.at[pl.ds(base, _K)], dst_v)
        _edge_flat(src_v, dst_v, f_v)
        pltpu.async_copy(adj_ref.at[f_v], g_v, sem).wait()

        def conv(i, carry2):
            sl = pl.ds(_woff(i), 16)
            gi_v[sl] = g_v[sl].astype(jnp.int32)
            return carry2
        lax.fori_loop(0, _NWIN, conv, 0)
        pltpu.sync_copy(ones_v, cnt_sp.at[gi_v], add=True)
        pltpu.sync_copy(gi_v, g_out.at[pl.ds(base, _K)])
        return carry

    lax.fori_loop(0, _T, chunk, 0)
    plsc.subcore_barrier()

    def dump(j, carry):
        pltpu.sync_copy(cnt_sp.at[pl.ds(sid * _CNT_STRIPE + j * 2000, 2000)], zv)
        pltpu.sync_copy(zv, cnt_out.at[cid, sid, j, 0])
        return carry
    lax.fori_loop(0, _CNT_STRIPE // 2000, dump, 0)


def _cnt_stage(src, dst, adj_ref):
    k = pl.kernel(
        _cnt_body,
        out_type=[
            jax.ShapeDtypeStruct((_NC, _NS, _CNT_STRIPE // 2000, 1, 2000), jnp.float32),
            jax.ShapeDtypeStruct((E,), jnp.int32),
        ],
        mesh=_sc_mesh(),
        compiler_params=pltpu.CompilerParams(needs_layout_passes=False),
        scratch_types=[
            pltpu.VMEM((_K,), jnp.int32),
            pltpu.VMEM((_K,), jnp.int32),
            pltpu.VMEM((_K,), jnp.int32),
            pltpu.VMEM((_K,), jnp.float32),
            pltpu.VMEM((_K,), jnp.int32),
            pltpu.VMEM((_K,), jnp.float32),
            pltpu.VMEM((2000,), jnp.float32),
            pltpu.VMEM_SHARED((E,), jnp.float32),
            pltpu.SemaphoreType.DMA,
        ],
    )
    return k(src, dst, adj_ref)


def _fin_body(src_hbm, dst_hbm, g_hbm, c0_hbm, c1_hbm, adj_ref,
              src_v, dst_v, f_v, gi_v, c0_v, c1_v, vals_v, sem):
    cid = lax.axis_index("c")
    sid = lax.axis_index("s")
    wid = sid * _NC + cid

    def chunk(t, carry):
        base = pl.multiple_of(wid * _EP + t * _K, 8)
        pltpu.sync_copy(src_hbm.at[pl.ds(base, _K)], src_v)
        pltpu.sync_copy(dst_hbm.at[pl.ds(base, _K)], dst_v)
        _edge_flat(src_v, dst_v, f_v)
        pltpu.sync_copy(g_hbm.at[pl.ds(base, _K)], gi_v)
        a = pltpu.async_copy(c0_hbm.at[gi_v], c0_v, sem)
        b = pltpu.async_copy(c1_hbm.at[gi_v], c1_v, sem)
        a.wait()
        b.wait()

        def add(i, carry2):
            sl = pl.ds(_woff(i), 16)
            vals_v[sl] = c0_v[sl] + c1_v[sl]
            return carry2
        lax.fori_loop(0, _NWIN, add, 0)
        pltpu.sync_copy(vals_v, adj_ref.at[f_v])
        return carry

    lax.fori_loop(0, _T, chunk, 0)


def _fin_stage(src, dst, gids, cnt0, cnt1, adj_ref):
    k = pl.kernel(
        _fin_body,
        out_type=(),
        mesh=_sc_mesh(),
        compiler_params=pltpu.CompilerParams(needs_layout_passes=False),
        scratch_types=[
            pltpu.VMEM((_K,), jnp.int32),
            pltpu.VMEM((_K,), jnp.int32),
            pltpu.VMEM((_K,), jnp.int32),
            pltpu.VMEM((_K,), jnp.int32),
            pltpu.VMEM((_K,), jnp.float32),
            pltpu.VMEM((_K,), jnp.float32),
            pltpu.VMEM((_K,), jnp.float32),
            pltpu.SemaphoreType.DMA,
        ],
    )
    return k(src, dst, gids, cnt0, cnt1, adj_ref)


_BLK = 1000


def _dense_body(x_ref, agg0_ref, agg1_ref, wrelT_ref, wrootT_ref, wmlpT_ref,
                brel_ref, bmlp_ref, s_ref, spad_ref, ssT_ref):
    i = pl.program_id(0)
    agg = agg0_ref[...] + agg1_ref[...]
    h = jnp.dot(agg, wrelT_ref[...], preferred_element_type=jnp.float32)
    h = h + brel_ref[...]
    h = h + jnp.dot(x_ref[...], wrootT_ref[...], preferred_element_type=jnp.float32)
    h = jnp.maximum(h, 0.0)
    sl = jnp.dot(h, wmlpT_ref[...], preferred_element_type=jnp.float32) + bmlp_ref[...]
    m = jnp.max(sl, axis=-1, keepdims=True)
    e = jnp.exp(sl - m)
    p = e / jnp.sum(e, axis=-1, keepdims=True)
    s_ref[...] = p
    spad_ref[...] = jnp.concatenate(
        [p, jnp.zeros((_BLK, F - C), jnp.float32)], axis=1)
    blk = jax.lax.dot_general(p, p, (((0,), (0,)), ((), ())),
                              preferred_element_type=jnp.float32)

    @pl.when(i == 0)
    def _():
        ssT_ref[...] = blk

    @pl.when(i != 0)
    def _():
        ssT_ref[...] += blk


def _dense_stage(x, agg0, agg1, W1_rel, b1_rel, W1_root, W_mlp, b_mlp):
    grid = (N // _BLK,)
    full = lambda shape: pl.BlockSpec(shape, lambda i: (0, 0))
    row = lambda shape: pl.BlockSpec(shape, lambda i: (i, 0))
    return pl.pallas_call(
        _dense_body,
        grid=grid,
        in_specs=[
            row((_BLK, F)),
            row((_BLK, F)),
            row((_BLK, F)),
            full((F, H)),
            full((F, H)),
            full((H, C)),
            full((1, H)),
            full((1, C)),
        ],
        out_specs=[row((_BLK, C)), row((_BLK, F)), full((C, C))],
        out_shape=[
            jax.ShapeDtypeStruct((N, C), jnp.float32),
            jax.ShapeDtypeStruct((N, F), jnp.float32),
            jax.ShapeDtypeStruct((C, C), jnp.float32),
        ],
    )(x, agg0, agg1, W1_rel.T, W1_root.T, W_mlp.T, b1_rel.reshape(1, H),
      b_mlp.reshape(1, C))


def _scalar_body(ssT_ref, nd_ref, mc_ref, o_ref):
    ss = ssT_ref[...]
    ssn = jnp.sqrt(jnp.sum(ss * ss))
    r = jax.lax.broadcasted_iota(jnp.int32, (C, C), 0)
    c = jax.lax.broadcasted_iota(jnp.int32, (C, C), 1)
    eye = jnp.where(r == c, 1.0, 0.0)
    diff = ss / ssn - eye / jnp.sqrt(jnp.float32(C))
    o_ref[...] = jnp.sqrt(jnp.sum(diff * diff)).reshape(1, 1)
    nd = nd_ref[...]
    num = jnp.sum(nd[:, :16])
    den = jnp.sum(nd[:, 16:])
    mc_ref[...] = (-(num / den)).reshape(1, 1)


def _scalar_stage(ssT, nd):
    mc, o = pl.pallas_call(
        _scalar_body,
        out_shape=[
            jax.ShapeDtypeStruct((1, 1), jnp.float32),
            jax.ShapeDtypeStruct((1, 1), jnp.float32),
        ],
    )(ssT, nd.reshape(_NW, 32))
    return mc.reshape(()), o.reshape(())


def kernel(x, edge_index, edge_weight, W1_rel, b1_rel, W1_root, W_mlp, b_mlp):
    src = edge_index[0]
    dst = edge_index[1]

    zrow = jnp.zeros((_NROWS, F), jnp.float32)
    aggp = _agg_stage(src, dst, edge_weight, x, zrow)

    s_soft, s_pad, ssT = _dense_stage(x, aggp[0, :N], aggp[1, :N], W1_rel,
                                      b1_rel, W1_root, W_mlp, b_mlp)

    adj_ref = jax.new_ref(_zero_stage())
    nd = _ids_stage(src, dst, s_pad, adj_ref)
    cnt, gids = _cnt_stage(src, dst, adj_ref)
    cnt = cnt.reshape(_NC, E)
    _fin_stage(src, dst, gids, cnt[0], cnt[1], adj_ref)
    adj = jax.freeze(adj_ref)

    mc_loss, o_loss = _scalar_stage(ssT, nd)

    return (s_soft, mc_loss, o_loss, adj.reshape(1, N, N))
